# baseline (speedup 1.0000x reference)
import jax
import jax.numpy as jnp
from jax.experimental import pallas as pl
from jax.experimental.pallas import tpu as pltpu

N_TOK = 8192
HIDDEN = 4096
KV_DIM = 256
OUT_DIM = 512
N_SLOTS = 65536
COMPRESS_RATIO = 4
TB = 256


def _proj_kernel(hs_ref, w_ref, posf_ref, ape_ref, kv_ref, sv_ref):
    acc = jax.lax.dot_general(
        hs_ref[...], w_ref[...],
        (((1,), (1,)), ((), ())),
        preferred_element_type=jnp.float32,
    )
    kv = acc[:, :KV_DIM]
    score = acc[:, KV_DIM:]
    posf = posf_ref[...]
    phase = posf - 4.0 * jnp.floor(posf * 0.25)
    iota8 = jax.lax.broadcasted_iota(jnp.int32, (1, 8), 1).astype(jnp.float32)
    onehot = (phase == iota8)
    pe = jax.lax.dot_general(
        onehot.astype(jnp.float32), ape_ref[...],
        (((1,), (0,)), ((), ())),
        preferred_element_type=jnp.float32,
    )
    kv_ref[...] = kv
    sv_ref[...] = jnp.concatenate([kv + pe, score], axis=1)


def _scatter_kernel(loc_ref, sv_ref, cache_ref, out_ref):
    del loc_ref, cache_ref
    out_ref[...] = sv_ref[...]


def kernel(hidden_states, positions, out_cache_loc, state_cache, weight, ape):
    posf = positions.astype(jnp.float32).reshape(N_TOK, 1)
    ape_pad = jnp.zeros((8, KV_DIM), jnp.float32).at[:COMPRESS_RATIO].set(ape)

    kv, slot_vals = pl.pallas_call(
        _proj_kernel,
        grid=(N_TOK // TB,),
        in_specs=[
            pl.BlockSpec((TB, HIDDEN), lambda i: (i, 0)),
            pl.BlockSpec((OUT_DIM, HIDDEN), lambda i: (0, 0)),
            pl.BlockSpec((TB, 1), lambda i: (i, 0)),
            pl.BlockSpec((8, KV_DIM), lambda i: (0, 0)),
        ],
        out_specs=[
            pl.BlockSpec((TB, KV_DIM), lambda i: (i, 0)),
            pl.BlockSpec((TB, OUT_DIM), lambda i: (i, 0)),
        ],
        out_shape=[
            jax.ShapeDtypeStruct((N_TOK, KV_DIM), jnp.float32),
            jax.ShapeDtypeStruct((N_TOK, OUT_DIM), jnp.float32),
        ],
    )(hidden_states, weight, posf, ape_pad)

    sv3 = slot_vals.reshape(N_TOK, 1, OUT_DIM)
    cache3 = state_cache.reshape(N_SLOTS, 1, OUT_DIM)
    new_cache = pl.pallas_call(
        _scatter_kernel,
        grid_spec=pltpu.PrefetchScalarGridSpec(
            num_scalar_prefetch=1,
            grid=(N_TOK,),
            in_specs=[
                pl.BlockSpec((1, 1, OUT_DIM), lambda i, loc: (i, 0, 0)),
                pl.BlockSpec((1, 1, OUT_DIM), lambda i, loc: (loc[i], 0, 0)),
            ],
            out_specs=pl.BlockSpec((1, 1, OUT_DIM), lambda i, loc: (loc[i], 0, 0)),
        ),
        out_shape=jax.ShapeDtypeStruct((N_SLOTS, 1, OUT_DIM), jnp.float32),
        input_output_aliases={2: 0},
    )(out_cache_loc, sv3, cache3)

    score = slot_vals[:, KV_DIM:]
    return kv, score, new_cache.reshape(N_SLOTS, OUT_DIM)
